# baseline (speedup 1.0000x reference)
import functools

import numpy as np
import jax
import jax.numpy as jnp
from jax import lax
from jax.experimental import pallas as pl
from jax.experimental.pallas import tpu as pltpu

_K0 = 0
_K1 = 12345
_KS2 = (_K0 ^ _K1 ^ 0x1BD11BDA) & 0xFFFFFFFF
_ROT = (13, 15, 26, 6, 17, 29, 16, 24)
_SCHED = (
    (_ROT[0:4], _K1, (_KS2 + 1) & 0xFFFFFFFF),
    (_ROT[4:8], _KS2, (_K0 + 2) & 0xFFFFFFFF),
    (_ROT[0:4], _K0, (_K1 + 3) & 0xFFFFFFFF),
    (_ROT[4:8], _K1, (_KS2 + 4) & 0xFFFFFFFF),
    (_ROT[0:4], _KS2, (_K0 + 5) & 0xFFFFFFFF),
)
_TINY = float(np.finfo(np.float32).tiny)


def _i32(v):
    return jnp.int32(np.uint32(v).astype(np.int32))


def _lrs(x, r):
    return lax.shift_right_logical(x, jnp.full(x.shape, r, jnp.int32))


def _threefry_bits(p):
    x0 = jnp.zeros_like(p)
    x1 = p + _i32(_K1)
    for rots, ka, kb in _SCHED:
        for r in rots:
            x0 = x0 + x1
            x1 = (x1 << r) | _lrs(x1, 32 - r)
            x1 = x1 ^ x0
        x0 = x0 + _i32(ka)
        x1 = x1 + _i32(kb)
    return x0 ^ x1


def _neglog_u(bits):
    fb = _lrs(bits, 9) | _i32(0x3F800000)
    f = lax.bitcast_convert_type(fb, jnp.float32) - jnp.float32(1.0)
    u = jnp.maximum(f, jnp.float32(_TINY))
    return -jnp.log(u)


def _sampler_body(N, LW, NI, R, pw_ref, alpha_ref, idx_ref, w_ref, c_scr,
                  c_big):
    b = pl.program_id(0)
    t = pl.program_id(1)
    CS = R * 128

    pw = pw_ref[0]
    m = jnp.max(pw)
    lse = m + jnp.log(jnp.sum(jnp.exp(pw - m)))
    pwn = pw - lse
    a = alpha_ref[0]
    lu = jnp.float32(-np.log(float(N)))
    t1 = pwn + jnp.log(a)
    t2 = lu + jnp.log(jnp.float32(1.0) - a)
    mm = jnp.maximum(t1, t2)
    q0 = mm + jnp.log(jnp.exp(t1 - mm) + jnp.exp(t2 - mm))
    m2 = jnp.max(q0)
    lse2 = m2 + jnp.log(jnp.sum(jnp.exp(q0 - m2)))
    qm = q0 - lse2
    mix = a < jnp.float32(1.0)
    q = jnp.where(mix, qm, pwn)
    wsrc = jnp.where(mix, pwn - qm, jnp.full_like(pwn, lu))
    c_scr[...] = jnp.exp(-q)
    w_ref[0] = wsrc
    for jj in range(NI):
        c_big[:, jj * LW:(jj + 1) * LW] = jnp.broadcast_to(
            c_scr[pl.ds(jj, 1), :], (R, LW))

    lane = lax.broadcasted_iota(jnp.int32, (R, LW), 1)
    lane128 = lax.broadcasted_iota(jnp.int32, (R, 128), 1)
    riota = lax.broadcasted_iota(jnp.int32, (R, LW), 0)
    bNN = b * _i32((N * N) & 0xFFFFFFFF)
    big = jnp.full((R, LW), jnp.int32(2**31 - 1))

    def outer(st, res):
        ib = bNN + (t * CS + riota * 128 + st) * N

        def inner(j, carry):
            best_s, best_j = carry
            ptile = ib + j * LW + lane
            E = _neglog_u(_threefry_bits(ptile))
            ctile = c_big[:, pl.ds(j * LW, LW)]
            s = E * ctile
            upd = s < best_s
            return (jnp.where(upd, s, best_s), jnp.where(upd, j, best_j))

        best_s, best_j = lax.fori_loop(
            0, NI, inner,
            (jnp.full((R, LW), jnp.inf, jnp.float32),
             jnp.zeros((R, LW), jnp.int32)),
            unroll=2)
        best_k = best_j * LW + lane
        mrow = jnp.min(best_s, axis=1, keepdims=True)
        cand = jnp.where(best_s == mrow, best_k, big)
        idxr = jnp.min(cand, axis=1, keepdims=True)
        return jnp.where(lane128 == st, idxr, res)

    res = lax.fori_loop(0, 128, outer, jnp.zeros((R, 128), jnp.int32))
    idx_ref[0, 0] = res + b * _i32(N)


def _make_sampler(B, N, T, LW):
    NI = N // LW
    CS = N // T
    R = CS // 128
    body = functools.partial(_sampler_body, N, LW, NI, R)
    return pl.pallas_call(
        body,
        grid=(B, T),
        in_specs=[
            pl.BlockSpec((1, NI, LW), lambda b, t: (b, 0, 0)),
            pl.BlockSpec(memory_space=pltpu.SMEM),
        ],
        out_specs=[
            pl.BlockSpec((1, 1, R, 128), lambda b, t: (b, t, 0, 0)),
            pl.BlockSpec((1, NI, LW), lambda b, t: (b, 0, 0)),
        ],
        out_shape=[
            jax.ShapeDtypeStruct((B, T, R, 128), jnp.int32),
            jax.ShapeDtypeStruct((B, NI, LW), jnp.float32),
        ],
        scratch_shapes=[
            pltpu.VMEM((NI, LW), jnp.float32),
            pltpu.VMEM((R, N), jnp.float32),
        ],
        compiler_params=pltpu.CompilerParams(
            dimension_semantics=("parallel", "arbitrary")),
    )


def kernel(particle_states, particle_weights, alpha):
    B, N = particle_weights.shape
    T = 8 if (N % (8 * 128) == 0 and N // 8 >= 128) else 1
    LW = 1024 if N % 1024 == 0 else N
    pw3 = particle_weights.reshape(B, N // LW, LW)
    a1 = jnp.asarray(alpha, jnp.float32).reshape(1)
    idx4, w3 = _make_sampler(B, N, T, LW)(pw3, a1)
    flat_idx = idx4.reshape(B * N)
    flat_states = particle_states.reshape(B * N, 3)
    new_states = jnp.take(flat_states, flat_idx, axis=0).reshape(B, N, 3)
    new_w = jnp.take(w3.reshape(B * N), flat_idx, axis=0).reshape(B, N)
    return (new_states, new_w)


# speedup vs baseline: 1.0069x; 1.0069x over previous
import functools

import numpy as np
import jax
import jax.numpy as jnp
from jax import lax
from jax.experimental import pallas as pl
from jax.experimental.pallas import tpu as pltpu
from jax.experimental.pallas import tpu_sc as plsc

_K0 = 0
_K1 = 12345
_KS2 = (_K0 ^ _K1 ^ 0x1BD11BDA) & 0xFFFFFFFF
_ROT = (13, 15, 26, 6, 17, 29, 16, 24)
_SCHED = (
    (_ROT[0:4], _K1, (_KS2 + 1) & 0xFFFFFFFF),
    (_ROT[4:8], _KS2, (_K0 + 2) & 0xFFFFFFFF),
    (_ROT[0:4], _K0, (_K1 + 3) & 0xFFFFFFFF),
    (_ROT[4:8], _K1, (_KS2 + 4) & 0xFFFFFFFF),
    (_ROT[0:4], _KS2, (_K0 + 5) & 0xFFFFFFFF),
)
_TINY = float(np.finfo(np.float32).tiny)


def _i32(v):
    return jnp.int32(np.uint32(v).astype(np.int32))


def _lrs(x, r):
    return lax.shift_right_logical(x, jnp.full(x.shape, r, jnp.int32))


def _threefry_bits(p):
    x0 = jnp.zeros_like(p)
    x1 = p + _i32(_K1)
    for rots, ka, kb in _SCHED:
        for r in rots:
            x0 = x0 + x1
            x1 = (x1 << r) | _lrs(x1, 32 - r)
            x1 = x1 ^ x0
        x0 = x0 + _i32(ka)
        x1 = x1 + _i32(kb)
    return x0 ^ x1


def _neglog_u(bits):
    fb = _lrs(bits, 9) | _i32(0x3F800000)
    f = lax.bitcast_convert_type(fb, jnp.float32) - jnp.float32(1.0)
    u = jnp.maximum(f, jnp.float32(_TINY))
    return -jnp.log(u)


def _sampler_body(N, LW, NI, R, pw_ref, alpha_ref, idx_ref, w_ref, c_scr,
                  c_big):
    b = pl.program_id(0)
    t = pl.program_id(1)
    CS = R * 128

    pw = pw_ref[0]
    m = jnp.max(pw)
    lse = m + jnp.log(jnp.sum(jnp.exp(pw - m)))
    pwn = pw - lse
    a = alpha_ref[0]
    lu = jnp.float32(-np.log(float(N)))
    t1 = pwn + jnp.log(a)
    t2 = lu + jnp.log(jnp.float32(1.0) - a)
    mm = jnp.maximum(t1, t2)
    q0 = mm + jnp.log(jnp.exp(t1 - mm) + jnp.exp(t2 - mm))
    m2 = jnp.max(q0)
    lse2 = m2 + jnp.log(jnp.sum(jnp.exp(q0 - m2)))
    qm = q0 - lse2
    mix = a < jnp.float32(1.0)
    q = jnp.where(mix, qm, pwn)
    wsrc = jnp.where(mix, pwn - qm, jnp.full_like(pwn, lu))
    c_scr[...] = jnp.exp(-q)
    w_ref[0] = wsrc
    for jj in range(NI):
        c_big[:, jj * LW:(jj + 1) * LW] = jnp.broadcast_to(
            c_scr[pl.ds(jj, 1), :], (R, LW))

    lane = lax.broadcasted_iota(jnp.int32, (R, LW), 1)
    lane128 = lax.broadcasted_iota(jnp.int32, (R, 128), 1)
    riota = lax.broadcasted_iota(jnp.int32, (R, LW), 0)
    bNN = b * _i32((N * N) & 0xFFFFFFFF)
    big = jnp.full((R, LW), jnp.int32(2**31 - 1))

    def outer(st, res):
        ib = bNN + (t * CS + riota * 128 + st) * N

        def inner(j, carry):
            best_s, best_j = carry
            ptile = ib + j * LW + lane
            E = _neglog_u(_threefry_bits(ptile))
            ctile = c_big[:, pl.ds(j * LW, LW)]
            s = E * ctile
            upd = s < best_s
            return (jnp.where(upd, s, best_s), jnp.where(upd, j, best_j))

        best_s, best_j = lax.fori_loop(
            0, NI, inner,
            (jnp.full((R, LW), jnp.inf, jnp.float32),
             jnp.zeros((R, LW), jnp.int32)),
            unroll=4)
        best_k = best_j * LW + lane
        mrow = jnp.min(best_s, axis=1, keepdims=True)
        cand = jnp.where(best_s == mrow, best_k, big)
        idxr = jnp.min(cand, axis=1, keepdims=True)
        return jnp.where(lane128 == st, idxr, res)

    res = lax.fori_loop(0, 128, outer, jnp.zeros((R, 128), jnp.int32))
    idx_ref[0, 0] = res + b * _i32(N)


def _make_sampler(B, N, T, LW):
    NI = N // LW
    CS = N // T
    R = CS // 128
    body = functools.partial(_sampler_body, N, LW, NI, R)
    return pl.pallas_call(
        body,
        grid=(B, T),
        in_specs=[
            pl.BlockSpec((1, NI, LW), lambda b, t: (b, 0, 0)),
            pl.BlockSpec(memory_space=pltpu.SMEM),
        ],
        out_specs=[
            pl.BlockSpec((1, 1, R, 128), lambda b, t: (b, t, 0, 0)),
            pl.BlockSpec((1, NI, LW), lambda b, t: (b, 0, 0)),
        ],
        out_shape=[
            jax.ShapeDtypeStruct((B, T, R, 128), jnp.int32),
            jax.ShapeDtypeStruct((B, NI, LW), jnp.float32),
        ],
        scratch_shapes=[
            pltpu.VMEM((NI, LW), jnp.float32),
            pltpu.VMEM((R, N), jnp.float32),
        ],
        compiler_params=pltpu.CompilerParams(
            dimension_semantics=("parallel", "arbitrary")),
    )


def _make_sc_gather(G, CH, D):
    info = plsc.get_sparse_core_info()
    NC, NS = info.num_cores, info.num_subcores
    NW = NC * NS

    @functools.partial(
        pl.kernel,
        mesh=plsc.VectorSubcoreMesh(core_axis_name="c", subcore_axis_name="s"),
        out_type=jax.ShapeDtypeStruct((NW, CH, 128, D), jnp.float32),
        scratch_types=[
            pltpu.VMEM((CH, 128), jnp.int32),
            pltpu.VMEM((128, D), jnp.float32),
            pltpu.VMEM((128, D), jnp.float32),
            pltpu.SemaphoreType.DMA,
            pltpu.SemaphoreType.DMA,
        ],
        compiler_params=pltpu.CompilerParams(use_tc_tiling_on_sc=False),
    )
    def gk(table_hbm, idx_hbm, out_hbm, idx_v, buf0, buf1, sem0, sem1):
        w = lax.axis_index("s") * NC + lax.axis_index("c")
        pltpu.sync_copy(idx_hbm.at[w], idx_v)

        def body(jj, carry):
            j0 = jj * 2
            c0 = pltpu.async_copy(table_hbm.at[idx_v.at[j0]], buf0, sem0)
            c1 = pltpu.async_copy(table_hbm.at[idx_v.at[j0 + 1]], buf1, sem1)
            c0.wait()
            pltpu.sync_copy(buf0, out_hbm.at[w, j0])
            c1.wait()
            pltpu.sync_copy(buf1, out_hbm.at[w, j0 + 1])
            return carry

        lax.fori_loop(0, CH // 2, body, 0)

    return gk


def kernel(particle_states, particle_weights, alpha):
    B, N = particle_weights.shape
    T = 8 if (N % (8 * 128) == 0 and N // 8 >= 128) else 1
    LW = 1024 if N % 1024 == 0 else N
    pw3 = particle_weights.reshape(B, N // LW, LW)
    a1 = jnp.asarray(alpha, jnp.float32).reshape(1)
    idx4, w3 = _make_sampler(B, N, T, LW)(pw3, a1)
    G = B * N
    D = 8
    flat_states = particle_states.reshape(G, 3)
    table = jnp.concatenate(
        [flat_states, w3.reshape(G, 1), jnp.zeros((G, D - 4), jnp.float32)],
        axis=1)
    NW = 32
    CH = G // NW // 128
    idx3 = idx4.reshape(NW, CH, 128)
    out4 = _make_sc_gather(G, CH, D)(table, idx3)
    flat_out = out4.reshape(G, D)
    new_states = flat_out[:, :3].reshape(B, N, 3)
    new_w = flat_out[:, 3].reshape(B, N)
    return (new_states, new_w)


# speedup vs baseline: 1.1732x; 1.1652x over previous
import functools

import numpy as np
import jax
import jax.numpy as jnp
from jax import lax
from jax.experimental import pallas as pl
from jax.experimental.pallas import tpu as pltpu
from jax.experimental.pallas import tpu_sc as plsc

_K0 = 0
_K1 = 12345
_KS2 = (_K0 ^ _K1 ^ 0x1BD11BDA) & 0xFFFFFFFF
_ROT = (13, 15, 26, 6, 17, 29, 16, 24)
_SCHED = (
    (_ROT[0:4], _K1, (_KS2 + 1) & 0xFFFFFFFF),
    (_ROT[4:8], _KS2, (_K0 + 2) & 0xFFFFFFFF),
    (_ROT[0:4], _K0, (_K1 + 3) & 0xFFFFFFFF),
    (_ROT[4:8], _K1, (_KS2 + 4) & 0xFFFFFFFF),
    (_ROT[0:4], _KS2, (_K0 + 5) & 0xFFFFFFFF),
)
_TINY = float(np.finfo(np.float32).tiny)


def _i32(v):
    return jnp.int32(np.uint32(v).astype(np.int32))


def _lrs(x, r):
    return lax.shift_right_logical(x, jnp.full(x.shape, r, jnp.int32))


def _threefry_bits(p):
    x0 = jnp.zeros_like(p)
    x1 = p + _i32(_K1)
    for rots, ka, kb in _SCHED:
        for r in rots:
            x0 = x0 + x1
            x1 = (x1 << r) | _lrs(x1, 32 - r)
            x1 = x1 ^ x0
        x0 = x0 + _i32(ka)
        x1 = x1 + _i32(kb)
    return x0 ^ x1


def _neglog_u(bits):
    fb = _lrs(bits, 9) | _i32(0x3F800000)
    f = lax.bitcast_convert_type(fb, jnp.float32) - jnp.float32(1.0)
    u = jnp.maximum(f, jnp.float32(_TINY))
    return -jnp.log(u)


def _sampler_body(N, LW, NI, R, CS, pw_ref, alpha_ref, idx_ref, w_ref, c_scr,
                  c_big):
    b = pl.program_id(0)
    t = pl.program_id(1)

    pw = pw_ref[0]
    m = jnp.max(pw)
    lse = m + jnp.log(jnp.sum(jnp.exp(pw - m)))
    pwn = pw - lse
    a = alpha_ref[0]
    lu = jnp.float32(-np.log(float(N)))
    t1 = pwn + jnp.log(a)
    t2 = lu + jnp.log(jnp.float32(1.0) - a)
    mm = jnp.maximum(t1, t2)
    q0 = mm + jnp.log(jnp.exp(t1 - mm) + jnp.exp(t2 - mm))
    m2 = jnp.max(q0)
    lse2 = m2 + jnp.log(jnp.sum(jnp.exp(q0 - m2)))
    qm = q0 - lse2
    mix = a < jnp.float32(1.0)
    q = jnp.where(mix, qm, pwn)
    wsrc = jnp.where(mix, pwn - qm, jnp.full_like(pwn, lu))
    c_scr[...] = jnp.exp(-q)
    w_ref[0] = wsrc
    for jj in range(NI):
        c_big[:, jj * LW:(jj + 1) * LW] = jnp.broadcast_to(
            c_scr[pl.ds(jj, 1), :], (R, LW))

    NST = CS // R
    NJ = N // LW
    lane = lax.broadcasted_iota(jnp.int32, (R, LW), 1)
    lane128 = lax.broadcasted_iota(jnp.int32, (8, 128), 1)
    riota = lax.broadcasted_iota(jnp.int32, (R, LW), 0)
    bNN = b * _i32((N * N) & 0xFFFFFFFF)
    big = jnp.full((R, LW), jnp.int32(2**31 - 1))
    pre_ib = bNN + (t * CS + (riota % 8) * 128 + (riota // 8) * NST) * N

    def outer(st, res):
        ib = pre_ib + st * N

        def inner(j, carry):
            best_s, best_j = carry
            ptile = ib + j * LW + lane
            E = _neglog_u(_threefry_bits(ptile))
            ctile = c_big[:, pl.ds(j * LW, LW)]
            s = E * ctile
            upd = s < best_s
            return (jnp.where(upd, s, best_s), jnp.where(upd, j, best_j))

        best_s, best_j = lax.fori_loop(
            0, NJ, inner,
            (jnp.full((R, LW), jnp.inf, jnp.float32),
             jnp.zeros((R, LW), jnp.int32)),
            unroll=4)
        best_k = best_j * LW + lane
        mrow = jnp.min(best_s, axis=1, keepdims=True)
        cand = jnp.where(best_s == mrow, best_k, big)
        idxr = jnp.min(cand, axis=1, keepdims=True)
        for g in range(R // 8):
            res = jnp.where(lane128 == st + g * NST,
                            idxr[g * 8:(g + 1) * 8], res)
        return res

    res = lax.fori_loop(0, NST, outer, jnp.zeros((8, 128), jnp.int32))
    idx_ref[0, 0] = res + b * _i32(N)


def _make_sampler(B, N, T, LW, R):
    NI = N // LW
    CS = N // T
    body = functools.partial(_sampler_body, N, LW, NI, R, CS)
    return pl.pallas_call(
        body,
        grid=(B, T),
        in_specs=[
            pl.BlockSpec((1, NI, LW), lambda b, t: (b, 0, 0)),
            pl.BlockSpec(memory_space=pltpu.SMEM),
        ],
        out_specs=[
            pl.BlockSpec((1, 1, 8, 128), lambda b, t: (b, t, 0, 0)),
            pl.BlockSpec((1, NI, LW), lambda b, t: (b, 0, 0)),
        ],
        out_shape=[
            jax.ShapeDtypeStruct((B, T, 8, 128), jnp.int32),
            jax.ShapeDtypeStruct((B, NI, LW), jnp.float32),
        ],
        scratch_shapes=[
            pltpu.VMEM((NI, LW), jnp.float32),
            pltpu.VMEM((R, N), jnp.float32),
        ],
        compiler_params=pltpu.CompilerParams(
            dimension_semantics=("parallel", "arbitrary")),
    )


def _make_sc_gather(G, CH, D):
    info = plsc.get_sparse_core_info()
    NC, NS = info.num_cores, info.num_subcores
    NW = NC * NS

    @functools.partial(
        pl.kernel,
        mesh=plsc.VectorSubcoreMesh(core_axis_name="c", subcore_axis_name="s"),
        out_type=jax.ShapeDtypeStruct((NW, CH, 128, D), jnp.float32),
        scratch_types=[
            pltpu.VMEM((CH, 128), jnp.int32),
            pltpu.VMEM((128, D), jnp.float32),
            pltpu.VMEM((128, D), jnp.float32),
            pltpu.SemaphoreType.DMA,
            pltpu.SemaphoreType.DMA,
        ],
        compiler_params=pltpu.CompilerParams(use_tc_tiling_on_sc=False),
    )
    def gk(table_hbm, idx_hbm, out_hbm, idx_v, buf0, buf1, sem0, sem1):
        w = lax.axis_index("s") * NC + lax.axis_index("c")
        pltpu.sync_copy(idx_hbm.at[w], idx_v)

        def body(jj, carry):
            j0 = jj * 2
            c0 = pltpu.async_copy(table_hbm.at[idx_v.at[j0]], buf0, sem0)
            c1 = pltpu.async_copy(table_hbm.at[idx_v.at[j0 + 1]], buf1, sem1)
            c0.wait()
            pltpu.sync_copy(buf0, out_hbm.at[w, j0])
            c1.wait()
            pltpu.sync_copy(buf1, out_hbm.at[w, j0 + 1])
            return carry

        lax.fori_loop(0, CH // 2, body, 0)

    return gk


def kernel(particle_states, particle_weights, alpha):
    B, N = particle_weights.shape
    T = 8 if (N % (8 * 128) == 0 and N // 8 >= 128) else 1
    LW = 256 if N % 256 == 0 else N
    R = 32
    pw3 = particle_weights.reshape(B, N // LW, LW)
    a1 = jnp.asarray(alpha, jnp.float32).reshape(1)
    idx4, w3 = _make_sampler(B, N, T, LW, R)(pw3, a1)
    G = B * N
    D = 8
    flat_states = particle_states.reshape(G, 3)
    table = jnp.concatenate(
        [flat_states, w3.reshape(G, 1), jnp.zeros((G, D - 4), jnp.float32)],
        axis=1)
    NW = 32
    CH = G // NW // 128
    idx3 = idx4.reshape(NW, CH, 128)
    out4 = _make_sc_gather(G, CH, D)(table, idx3)
    flat_out = out4.reshape(G, D)
    new_states = flat_out[:, :3].reshape(B, N, 3)
    new_w = flat_out[:, 3].reshape(B, N)
    return (new_states, new_w)


# speedup vs baseline: 1.1756x; 1.0020x over previous
import functools

import numpy as np
import jax
import jax.numpy as jnp
from jax import lax
from jax.experimental import pallas as pl
from jax.experimental.pallas import tpu as pltpu
from jax.experimental.pallas import tpu_sc as plsc

_K0 = 0
_K1 = 12345
_KS2 = (_K0 ^ _K1 ^ 0x1BD11BDA) & 0xFFFFFFFF
_ROT = (13, 15, 26, 6, 17, 29, 16, 24)
_SCHED = (
    (_ROT[0:4], _K1, (_KS2 + 1) & 0xFFFFFFFF),
    (_ROT[4:8], _KS2, (_K0 + 2) & 0xFFFFFFFF),
    (_ROT[0:4], _K0, (_K1 + 3) & 0xFFFFFFFF),
    (_ROT[4:8], _K1, (_KS2 + 4) & 0xFFFFFFFF),
    (_ROT[0:4], _KS2, (_K0 + 5) & 0xFFFFFFFF),
)
_TINY = float(np.finfo(np.float32).tiny)


def _i32(v):
    return jnp.int32(np.uint32(v).astype(np.int32))


def _lrs(x, r):
    return lax.shift_right_logical(x, jnp.full(x.shape, r, jnp.int32))


def _threefry_bits(p):
    x0 = jnp.zeros_like(p)
    x1 = p + _i32(_K1)
    for rots, ka, kb in _SCHED:
        for r in rots:
            x0 = x0 + x1
            x1 = (x1 << r) | _lrs(x1, 32 - r)
            x1 = x1 ^ x0
        x0 = x0 + _i32(ka)
        x1 = x1 + _i32(kb)
    return x0 ^ x1


def _neglog_u(bits):
    fb = _lrs(bits, 9) | _i32(0x3F800000)
    f = lax.bitcast_convert_type(fb, jnp.float32) - jnp.float32(1.0)
    u = jnp.maximum(f, jnp.float32(_TINY))
    return -jnp.log(u)


def _sampler_body(N, LW, NI, R, CS, pw_ref, alpha_ref, idx_ref, w_ref, c_scr,
                  c_big):
    b = pl.program_id(0)
    t = pl.program_id(1)

    pw = pw_ref[0]
    m = jnp.max(pw)
    lse = m + jnp.log(jnp.sum(jnp.exp(pw - m)))
    pwn = pw - lse
    a = alpha_ref[0]
    lu = jnp.float32(-np.log(float(N)))
    t1 = pwn + jnp.log(a)
    t2 = lu + jnp.log(jnp.float32(1.0) - a)
    mm = jnp.maximum(t1, t2)
    q0 = mm + jnp.log(jnp.exp(t1 - mm) + jnp.exp(t2 - mm))
    m2 = jnp.max(q0)
    lse2 = m2 + jnp.log(jnp.sum(jnp.exp(q0 - m2)))
    qm = q0 - lse2
    mix = a < jnp.float32(1.0)
    q = jnp.where(mix, qm, pwn)
    wsrc = jnp.where(mix, pwn - qm, jnp.full_like(pwn, lu))
    c_scr[...] = jnp.exp(-q)
    w_ref[0] = wsrc
    for jj in range(NI):
        c_big[:, jj * LW:(jj + 1) * LW] = jnp.broadcast_to(
            c_scr[pl.ds(jj, 1), :], (R, LW))

    NST = CS // R
    NJ = N // LW
    lane = lax.broadcasted_iota(jnp.int32, (R, LW), 1)
    lane128 = lax.broadcasted_iota(jnp.int32, (8, 128), 1)
    riota = lax.broadcasted_iota(jnp.int32, (R, LW), 0)
    bNN = b * _i32((N * N) & 0xFFFFFFFF)
    big = jnp.full((R, LW), jnp.int32(2**31 - 1))
    pre_ib = bNN + (t * CS + (riota % 8) * 128 + (riota // 8) * NST) * N

    def outer(st, res):
        ib = pre_ib + st * N

        def inner(j, carry):
            best_s, best_j = carry
            ptile = ib + j * LW + lane
            E = _neglog_u(_threefry_bits(ptile))
            ctile = c_big[:, pl.ds(j * LW, LW)]
            s = E * ctile
            upd = s < best_s
            return (jnp.where(upd, s, best_s), jnp.where(upd, j, best_j))

        best_s, best_j = lax.fori_loop(
            0, NJ, inner,
            (jnp.full((R, LW), jnp.inf, jnp.float32),
             jnp.zeros((R, LW), jnp.int32)),
            unroll=4)
        best_k = best_j * LW + lane
        mrow = jnp.min(best_s, axis=1, keepdims=True)
        cand = jnp.where(best_s == mrow, best_k, big)
        idxr = jnp.min(cand, axis=1, keepdims=True)
        for g in range(R // 8):
            res = jnp.where(lane128 == st + g * NST,
                            idxr[g * 8:(g + 1) * 8], res)
        return res

    res = lax.fori_loop(0, NST, outer, jnp.zeros((8, 128), jnp.int32),
                        unroll=2)
    idx_ref[0, 0] = res + b * _i32(N)


def _make_sampler(B, N, T, LW, R):
    NI = N // LW
    CS = N // T
    body = functools.partial(_sampler_body, N, LW, NI, R, CS)
    return pl.pallas_call(
        body,
        grid=(B, T),
        in_specs=[
            pl.BlockSpec((1, NI, LW), lambda b, t: (b, 0, 0)),
            pl.BlockSpec(memory_space=pltpu.SMEM),
        ],
        out_specs=[
            pl.BlockSpec((1, 1, 8, 128), lambda b, t: (b, t, 0, 0)),
            pl.BlockSpec((1, NI, LW), lambda b, t: (b, 0, 0)),
        ],
        out_shape=[
            jax.ShapeDtypeStruct((B, T, 8, 128), jnp.int32),
            jax.ShapeDtypeStruct((B, NI, LW), jnp.float32),
        ],
        scratch_shapes=[
            pltpu.VMEM((NI, LW), jnp.float32),
            pltpu.VMEM((R, N), jnp.float32),
        ],
        compiler_params=pltpu.CompilerParams(
            dimension_semantics=("parallel", "arbitrary")),
    )


def _make_sc_gather(G, CH, D):
    info = plsc.get_sparse_core_info()
    NC, NS = info.num_cores, info.num_subcores
    NW = NC * NS

    @functools.partial(
        pl.kernel,
        mesh=plsc.VectorSubcoreMesh(core_axis_name="c", subcore_axis_name="s"),
        out_type=jax.ShapeDtypeStruct((NW, CH, 128, D), jnp.float32),
        scratch_types=[
            pltpu.VMEM((CH, 128), jnp.int32),
            pltpu.VMEM((128, D), jnp.float32),
            pltpu.VMEM((128, D), jnp.float32),
            pltpu.SemaphoreType.DMA,
            pltpu.SemaphoreType.DMA,
        ],
        compiler_params=pltpu.CompilerParams(use_tc_tiling_on_sc=False),
    )
    def gk(table_hbm, idx_hbm, out_hbm, idx_v, buf0, buf1, sem0, sem1):
        w = lax.axis_index("s") * NC + lax.axis_index("c")
        pltpu.sync_copy(idx_hbm.at[w], idx_v)

        def body(jj, carry):
            j0 = jj * 2
            c0 = pltpu.async_copy(table_hbm.at[idx_v.at[j0]], buf0, sem0)
            c1 = pltpu.async_copy(table_hbm.at[idx_v.at[j0 + 1]], buf1, sem1)
            c0.wait()
            pltpu.sync_copy(buf0, out_hbm.at[w, j0])
            c1.wait()
            pltpu.sync_copy(buf1, out_hbm.at[w, j0 + 1])
            return carry

        lax.fori_loop(0, CH // 2, body, 0)

    return gk


def kernel(particle_states, particle_weights, alpha):
    B, N = particle_weights.shape
    T = 8 if (N % (8 * 128) == 0 and N // 8 >= 128) else 1
    LW = 256 if N % 256 == 0 else N
    R = 32
    pw3 = particle_weights.reshape(B, N // LW, LW)
    a1 = jnp.asarray(alpha, jnp.float32).reshape(1)
    idx4, w3 = _make_sampler(B, N, T, LW, R)(pw3, a1)
    G = B * N
    D = 8
    flat_states = particle_states.reshape(G, 3)
    table = jnp.concatenate(
        [flat_states, w3.reshape(G, 1), jnp.zeros((G, D - 4), jnp.float32)],
        axis=1)
    NW = 32
    CH = G // NW // 128
    idx3 = idx4.reshape(NW, CH, 128)
    out4 = _make_sc_gather(G, CH, D)(table, idx3)
    flat_out = out4.reshape(G, D)
    new_states = flat_out[:, :3].reshape(B, N, 3)
    new_w = flat_out[:, 3].reshape(B, N)
    return (new_states, new_w)
